# baseline (speedup 1.0000x reference)
import numpy as np
import jax
import jax.numpy as jnp
from jax import lax
from jax.experimental import pallas as pl
from jax.experimental.pallas import tpu as pltpu
from jax.experimental.pallas import tpu_sc as plsc

_B, _S, _R, _DM = 4, 8192, 4096, 768
_N = _B * _S
_NOUT = _B * _R
_NSPLIT = 4
_NPART = _NOUT // _NSPLIT

_NC, _NS = 2, 16
_NW = _NC * _NS
_CHUNK = 64
_ROWS_PER_W = 2 * _NPART // _NW
_NCHUNK = _ROWS_PER_W // _CHUNK

_TC_BLK = 1024
_NBLK_P = _NPART // _TC_BLK
_NBLK = _NOUT // _TC_BLK


def _static_plan():
    base = jax.random.key(1234)
    keeps, reds = [], []
    for i in range(_B):
        perm = jax.random.permutation(jax.random.fold_in(base, i), _S - 1)
        keep = jnp.concatenate(
            [jnp.sort(perm[: _R - 1]), jnp.array([_S - 1], dtype=perm.dtype)]
        ) + i * _S
        red = perm[_R - 1:] + i * _S
        keeps.append(keep)
        reds.append(red)
    ik = np.asarray(jnp.concatenate(keeps)).astype(np.int64)
    ir = np.asarray(jnp.concatenate(reds)).astype(np.int64)
    is_red = np.zeros(_N, dtype=bool)
    is_red[ir] = True
    need = (ik > 0) & is_red[np.maximum(ik - 1, 0)]
    pred = np.maximum(ik - 1, 0)
    idx_plans, masks = [], []
    for h in range(_NSPLIT):
        sl = slice(h * _NPART, (h + 1) * _NPART)
        all_idx = np.concatenate([ik[sl], pred[sl]]).astype(np.int32)
        idx_plans.append(all_idx.reshape(_NW, _NCHUNK, _CHUNK))
        masks.append(need[sl].astype(np.float32).reshape(_NPART, 1))
    return idx_plans, masks


_IDX_PLANS, _NEED_MASKS = _static_plan()


def _sc_gather_body(xf_hbm, idx_hbm, gp_hbm, idx_v, buf0, buf1, sg0, sg1, sw0, sw1):
    c = lax.axis_index("c")
    s = lax.axis_index("s")
    wid = s * _NC + c
    base = wid * _ROWS_PER_W
    bufs, sg, sw = (buf0, buf1), (sg0, sg1), (sw0, sw1)
    pltpu.sync_copy(idx_hbm.at[wid], idx_v)

    def gather(j):
        b = j % 2
        return pltpu.async_copy(xf_hbm.at[idx_v.at[j]], bufs[b], sg[b])

    def write(j):
        b = j % 2
        return pltpu.async_copy(
            bufs[b], gp_hbm.at[pl.ds(base + j * _CHUNK, _CHUNK)], sw[b])

    g = [None] * _NCHUNK
    w = [None] * _NCHUNK
    g[0] = gather(0)
    for j in range(1, _NCHUNK):
        if j >= 2:
            w[j - 2].wait()
        g[j] = gather(j)
        g[j - 1].wait()
        w[j - 1] = write(j - 1)
    w[_NCHUNK - 2].wait()
    g[_NCHUNK - 1].wait()
    w[_NCHUNK - 1] = write(_NCHUNK - 1)
    w[_NCHUNK - 1].wait()


def _sc_gather(xf, idx):
    mesh = plsc.VectorSubcoreMesh(core_axis_name="c", subcore_axis_name="s")
    fn = pl.kernel(
        _sc_gather_body,
        out_type=jax.ShapeDtypeStruct((2 * _NPART, _DM), jnp.float32),
        mesh=mesh,
        scratch_types=[
            pltpu.VMEM((_NCHUNK, _CHUNK), jnp.int32),
            pltpu.VMEM((_CHUNK, _DM), jnp.float32),
            pltpu.VMEM((_CHUNK, _DM), jnp.float32),
            pltpu.SemaphoreType.DMA,
            pltpu.SemaphoreType.DMA,
            pltpu.SemaphoreType.DMA,
            pltpu.SemaphoreType.DMA,
        ],
    )
    return fn(xf, idx)


def _tc_body_first(g_ref, p_ref, m_ref, w_ref, o_ref):
    p = p_ref[...] * m_ref[...]
    o_ref[...] = g_ref[...] + lax.dot_general(
        p, w_ref[...], (((1,), (1,)), ((), ())),
        preferred_element_type=jnp.float32,
    )


def _tc_body_next(prev_ref, g_ref, p_ref, m_ref, w_ref, o_ref):
    del prev_ref
    _tc_body_first(g_ref, p_ref, m_ref, w_ref, o_ref)


def _tc_merge(prev, gp, mask, W, part):
    gp_specs = [
        pl.BlockSpec((_TC_BLK, _DM), lambda i: (i, 0)),
        pl.BlockSpec((_TC_BLK, _DM), lambda i: (i + _NBLK_P, 0)),
        pl.BlockSpec((_TC_BLK, 1), lambda i: (i, 0)),
        pl.BlockSpec((_DM, _DM), lambda i: (0, 0)),
    ]
    out_spec = pl.BlockSpec(
        (_TC_BLK, _DM), lambda i, part=part: (i + part * _NBLK_P, 0))
    common = dict(
        grid=(_NBLK_P,),
        out_specs=out_spec,
        out_shape=jax.ShapeDtypeStruct((_NOUT, _DM), jnp.float32),
    )
    if prev is None:
        return pl.pallas_call(
            _tc_body_first, in_specs=gp_specs, **common)(gp, gp, mask, W)
    return pl.pallas_call(
        _tc_body_next,
        in_specs=[pl.BlockSpec((8, 128), lambda i: (0, 0))] + gp_specs,
        input_output_aliases={0: 0},
        **common)(prev, gp, gp, mask, W)


def kernel(x, W):
    xf = x.reshape(_N, _DM)
    gps = [_sc_gather(xf, jnp.asarray(_IDX_PLANS[h])) for h in range(_NSPLIT)]
    out = None
    for h in range(_NSPLIT):
        out = _tc_merge(out, gps[h], jnp.asarray(_NEED_MASKS[h]), W, h)
    return out.reshape(_B, _R, _DM)


# speedup vs baseline: 1.0353x; 1.0353x over previous
import numpy as np
import jax
import jax.numpy as jnp
from jax import lax
from jax.experimental import pallas as pl
from jax.experimental.pallas import tpu as pltpu
from jax.experimental.pallas import tpu_sc as plsc

_B, _S, _R, _DM = 4, 8192, 4096, 768
_N = _B * _S
_NOUT = _B * _R
_NSPLIT = 2
_NPART = _NOUT // _NSPLIT

_NC, _NS = 2, 16
_NW = _NC * _NS
_CHUNK = 64
_ROWS_PER_W = 2 * _NPART // _NW
_NCHUNK = _ROWS_PER_W // _CHUNK

_TC_BLK = 1024
_NBLK_P = _NPART // _TC_BLK
_NBLK = _NOUT // _TC_BLK


def _static_plan():
    base = jax.random.key(1234)
    keeps, reds = [], []
    for i in range(_B):
        perm = jax.random.permutation(jax.random.fold_in(base, i), _S - 1)
        keep = jnp.concatenate(
            [jnp.sort(perm[: _R - 1]), jnp.array([_S - 1], dtype=perm.dtype)]
        ) + i * _S
        red = perm[_R - 1:] + i * _S
        keeps.append(keep)
        reds.append(red)
    ik = np.asarray(jnp.concatenate(keeps)).astype(np.int64)
    ir = np.asarray(jnp.concatenate(reds)).astype(np.int64)
    is_red = np.zeros(_N, dtype=bool)
    is_red[ir] = True
    need = (ik > 0) & is_red[np.maximum(ik - 1, 0)]
    pred = np.maximum(ik - 1, 0)
    idx_plans, masks = [], []
    for h in range(_NSPLIT):
        sl = slice(h * _NPART, (h + 1) * _NPART)
        all_idx = np.concatenate([ik[sl], pred[sl]]).astype(np.int32)
        idx_plans.append(all_idx.reshape(_NW, _NCHUNK, _CHUNK))
        masks.append(need[sl].astype(np.float32).reshape(_NPART, 1))
    return idx_plans, masks


_IDX_PLANS, _NEED_MASKS = _static_plan()


def _sc_gather_body(xf_hbm, idx_hbm, gp_hbm, idx_v, buf0, buf1, sg0, sg1, sw0, sw1):
    c = lax.axis_index("c")
    s = lax.axis_index("s")
    wid = s * _NC + c
    base = wid * _ROWS_PER_W
    bufs, sg, sw = (buf0, buf1), (sg0, sg1), (sw0, sw1)
    pltpu.sync_copy(idx_hbm.at[wid], idx_v)

    def gather(j):
        b = j % 2
        return pltpu.async_copy(xf_hbm.at[idx_v.at[j]], bufs[b], sg[b])

    def write(j):
        b = j % 2
        return pltpu.async_copy(
            bufs[b], gp_hbm.at[pl.ds(base + j * _CHUNK, _CHUNK)], sw[b])

    g = [None] * _NCHUNK
    w = [None] * _NCHUNK
    g[0] = gather(0)
    for j in range(1, _NCHUNK):
        if j >= 2:
            w[j - 2].wait()
        g[j] = gather(j)
        g[j - 1].wait()
        w[j - 1] = write(j - 1)
    w[_NCHUNK - 2].wait()
    g[_NCHUNK - 1].wait()
    w[_NCHUNK - 1] = write(_NCHUNK - 1)
    w[_NCHUNK - 1].wait()


def _sc_gather(xf, idx):
    mesh = plsc.VectorSubcoreMesh(core_axis_name="c", subcore_axis_name="s")
    fn = pl.kernel(
        _sc_gather_body,
        out_type=jax.ShapeDtypeStruct((2 * _NPART, _DM), jnp.float32),
        mesh=mesh,
        scratch_types=[
            pltpu.VMEM((_NCHUNK, _CHUNK), jnp.int32),
            pltpu.VMEM((_CHUNK, _DM), jnp.float32),
            pltpu.VMEM((_CHUNK, _DM), jnp.float32),
            pltpu.SemaphoreType.DMA,
            pltpu.SemaphoreType.DMA,
            pltpu.SemaphoreType.DMA,
            pltpu.SemaphoreType.DMA,
        ],
    )
    return fn(xf, idx)


def _tc_body_first(g_ref, p_ref, m_ref, w_ref, o_ref):
    p = p_ref[...] * m_ref[...]
    o_ref[...] = g_ref[...] + lax.dot_general(
        p, w_ref[...], (((1,), (1,)), ((), ())),
        preferred_element_type=jnp.float32,
    )


def _tc_body_next(prev_ref, g_ref, p_ref, m_ref, w_ref, o_ref):
    del prev_ref
    _tc_body_first(g_ref, p_ref, m_ref, w_ref, o_ref)


def _tc_merge(prev, gp, mask, W, part):
    gp_specs = [
        pl.BlockSpec((_TC_BLK, _DM), lambda i: (i, 0)),
        pl.BlockSpec((_TC_BLK, _DM), lambda i: (i + _NBLK_P, 0)),
        pl.BlockSpec((_TC_BLK, 1), lambda i: (i, 0)),
        pl.BlockSpec((_DM, _DM), lambda i: (0, 0)),
    ]
    out_spec = pl.BlockSpec(
        (_TC_BLK, _DM), lambda i, part=part: (i + part * _NBLK_P, 0))
    common = dict(
        grid=(_NBLK_P,),
        out_specs=out_spec,
        out_shape=jax.ShapeDtypeStruct((_NOUT, _DM), jnp.float32),
    )
    if prev is None:
        return pl.pallas_call(
            _tc_body_first, in_specs=gp_specs, **common)(gp, gp, mask, W)
    return pl.pallas_call(
        _tc_body_next,
        in_specs=[pl.BlockSpec((8, 128), lambda i: (0, 0))] + gp_specs,
        input_output_aliases={0: 0},
        **common)(prev, gp, gp, mask, W)


def kernel(x, W):
    xf = x.reshape(_N, _DM)
    gps = [_sc_gather(xf, jnp.asarray(_IDX_PLANS[h])) for h in range(_NSPLIT)]
    out = None
    for h in range(_NSPLIT):
        out = _tc_merge(out, gps[h], jnp.asarray(_NEED_MASKS[h]), W, h)
    return out.reshape(_B, _R, _DM)
